# baseline (speedup 1.0000x reference)
import functools

import numpy as np
import jax
import jax.numpy as jnp
from jax import lax
from jax.experimental import pallas as pl
from jax.experimental.pallas import tpu as pltpu
from jax.experimental.pallas import tpu_sc as plsc

N = 10000
E = 320000
HID = 128
NH = 16
HD = HID // NH
NRG = 20
EF = 4
RFEAT = NRG * EF
R_MIN, R_MAX = 0.0, 10.0
_STEP = (R_MAX - R_MIN) / (NRG - 1)
_COEFF = -0.5 / _STEP**2
_OFFS = np.linspace(R_MIN, R_MAX, NRG).astype(np.float32)[None, :]
_ISQ = np.float32(1.0 / np.sqrt(HD))

_NC, _NS = 2, 16
_NW = _NC * _NS
_K = 128
_NCHUNK = E // _K
_ROUNDS = -(-_NCHUNK // _NW)
_ROUNDS16 = -(-_NCHUNK // _NS)

_TE = 2000
_TN = 2000


def _sel(n_in, n_out, fn):
    m = np.zeros((n_in, n_out), np.float32)
    for i in range(n_in):
        for j in range(n_out):
            if fn(i, j):
                m[i, j] = 1.0
    return m


_S_HD = _sel(HID, NH, lambda j, h: j // HD == h)
_S_WIDE = _sel(NH, HID, lambda h, j: j // HD == h)
_A4 = _sel(EF, RFEAT, lambda f, j: j // NRG == f)
_A20 = _sel(NRG, RFEAT, lambda g, j: j % NRG == g)
_E1 = _sel(NH, 3 * NH, lambda h, j: j // 3 == h)
_E2 = _sel(3, 3 * NH, lambda c, j: j % 3 == c)
_M48 = _sel(3 * NH, 3, lambda j, c: j % 3 == c) / NH


def _ln_relu(h1, g, be):
    mu = jnp.mean(h1, axis=-1, keepdims=True)
    var = jnp.mean((h1 - mu) ** 2, axis=-1, keepdims=True)
    hn = (h1 - mu) * lax.rsqrt(var + 1e-5) * g + be
    return jnp.maximum(hn, 0.0)


def _dot(a, b):
    return jnp.dot(a, b, preferred_element_type=jnp.float32)


def _bdot(a, b):
    return jnp.dot(a.astype(jnp.bfloat16), b, preferred_element_type=jnp.float32)


def _node_pre_body(h_ref, xp_ref, w1, b1, g, be, w2, b2, t1d, t1s):
    h = h_ref[:]
    xp = xp_ref[:]
    r = _ln_relu(_dot(h, w1[:]) + b1[:], g[:], be[:])
    q = _dot(r, w2[:]) + b2[:]
    t1d[:] = jnp.concatenate([h, q, xp], axis=-1)
    t1s[:] = jnp.concatenate([h, xp], axis=-1)


def _edge1_body(g1d, g1s, ea_ref,
                kw1, kb1, kg, kbe, kw2, kb2,
                vw1, vb1, vg, vbe, vw2, vb2,
                eww, ewb, a4, a20, shd, swide, offs,
                sca, scb):
    d = g1d[:]
    hi = d[:, 0:HID]
    qd = d[:, HID:2 * HID]
    s = g1s[:]
    hj = s[:, 0:HID]
    relp = d[:, 2 * HID:2 * HID + 16] - s[:, HID:HID + 16]
    rel = relp[:, 0:3]
    ea = ea_ref[:]
    dist = jnp.sqrt(jnp.sum(relp * relp, axis=-1, keepdims=True))
    df = jnp.exp(_COEFF * (dist - offs[:]) ** 2)
    rf = _dot(ea, a4[:]) * _dot(df, a20[:])
    kv84 = jnp.concatenate([ea, rf], axis=-1)
    h1k = (_bdot(kv84, kw1[0:84, :]) + _bdot(hi, kw1[84:212, :])
           + _bdot(hj, kw1[212:340, :]) + kb1[:])
    k = _bdot(_ln_relu(h1k, kg[:], kbe[:]), kw2[:]) + kb2[:]
    h1v = (_bdot(kv84, vw1[0:84, :]) + _bdot(hi, vw1[84:212, :])
           + _bdot(hj, vw1[212:340, :]) + vb1[:])
    v = _bdot(_ln_relu(h1v, vg[:], vbe[:]), vw2[:]) + vb2[:]
    ew = jax.nn.sigmoid(jnp.sum(rf * eww[:], axis=-1, keepdims=True) + ewb[:])
    v = v * ew
    w = jnp.exp(_dot(qd * k, shd[:]) * _ISQ)
    wv = _dot(w, swide[:]) * v
    sca[:] = wv
    zpad = jnp.zeros((w.shape[0], HID - NH - 4 - NRG), jnp.float32)
    scb[:] = jnp.concatenate([w, rel, dist, df, zpad], axis=-1)


def _node_mid_body(acc_ref, h_ref,
                   ow1, ob1, og, obe, ow2, ob2,
                   qw1, qb1, qg, qbe, qw2, qb2,
                   swide, nh_out, t2d):
    num = acc_ref[0]
    den = acc_ref[1][:, 0:NH]
    attn = num / (_dot(den, swide[:]) + 1e-16)
    h = h_ref[:]
    h1 = _dot(attn, ow1[0:HID, :]) + _dot(h, ow1[HID:2 * HID, :]) + ob1[:]
    o = _dot(_ln_relu(h1, og[:], obe[:]), ow2[:]) + ob2[:]
    nh = o + h
    r = _ln_relu(_dot(nh, qw1[:]) + qb1[:], qg[:], qbe[:])
    q2 = _dot(r, qw2[:]) + qb2[:]
    nh_out[:] = nh
    t2d[:] = jnp.concatenate([nh, q2], axis=-1)


def _edge2_body(g2d, g2s, ea_ref, rf_ref,
                kw1, kb1, kg, kbe, kw2, kb2,
                vw1, vb1, vg, vbe, vw2, vb2,
                eww, ewb, a4, a20, shd, e1, e2,
                sc2):
    d = g2d[:]
    hi = d[:, 0:HID]
    qd = d[:, HID:2 * HID]
    hj = g2s[:]
    ea = ea_ref[:]
    rfin = rf_ref[:]
    rel = rfin[:, NH:NH + 3]
    df = rfin[:, NH + 4:NH + 4 + NRG]
    rf = _dot(ea, a4[:]) * _dot(df, a20[:])
    kv84 = jnp.concatenate([ea, rf], axis=-1)
    h1k = (_bdot(kv84, kw1[0:84, :]) + _bdot(hi, kw1[84:212, :])
           + _bdot(hj, kw1[212:340, :]) + kb1[:])
    k = _bdot(_ln_relu(h1k, kg[:], kbe[:]), kw2[:]) + kb2[:]
    h1v = (_bdot(kv84, vw1[0:84, :]) + _bdot(hi, vw1[84:212, :])
           + _bdot(hj, vw1[212:340, :]) + vb1[:])
    v2 = _bdot(_ln_relu(h1v, vg[:], vbe[:]), vw2[:]) + vb2[:]
    ew = jax.nn.sigmoid(jnp.sum(rf * eww[:], axis=-1, keepdims=True) + ewb[:])
    v2 = v2 * ew
    w = jnp.exp(_dot(qd * k, shd[:]) * _ISQ)
    mv = w * v2
    sv = _dot(mv, e1[:]) * _dot(rel, e2[:])
    zpad = jnp.zeros((w.shape[0], HID - 4 * NH), jnp.float32)
    sc2[:] = jnp.concatenate([sv, w, zpad], axis=-1)


def _node_fin_body(acc_ref, e1, m48, delta):
    a = acc_ref[0] + acc_ref[1]
    num = a[:, 0:3 * NH]
    den = a[:, 3 * NH:4 * NH]
    ratio = num / (_dot(den, e1[:]) + 1e-16)
    delta[:] = _dot(ratio, m48[:])


def _full(shape):
    return pl.BlockSpec(shape, lambda i: (0,) * len(shape))


def _mlp_specs(din):
    return [_full((din, HID)), _full((1, HID)), _full((1, HID)),
            _full((1, HID)), _full((HID, HID)), _full((1, HID))]


def _mlp_args(p):
    return (p["W1"], p["b1"][None, :], p["g"][None, :], p["be"][None, :],
            p["W2"], p["b2"][None, :])


def _mlp_args_bf(p):
    return (p["W1"].astype(jnp.bfloat16), p["b1"][None, :], p["g"][None, :],
            p["be"][None, :], p["W2"].astype(jnp.bfloat16), p["b2"][None, :])


def _node_pre(h, xp, p):
    return pl.pallas_call(
        _node_pre_body,
        grid=(N // _TN,),
        in_specs=([pl.BlockSpec((_TN, HID), lambda i: (i, 0)),
                   pl.BlockSpec((_TN, HID), lambda i: (i, 0))]
                  + _mlp_specs(HID)),
        out_specs=[pl.BlockSpec((_TN, 3 * HID), lambda i: (i, 0)),
                   pl.BlockSpec((_TN, 2 * HID), lambda i: (i, 0))],
        out_shape=[jax.ShapeDtypeStruct((N, 3 * HID), jnp.float32),
                   jax.ShapeDtypeStruct((N, 2 * HID), jnp.float32)],
    )(h, xp, *_mlp_args(p))


def _edge1(g1d, g1s, ea, pk, pv, eww, ewb):
    return pl.pallas_call(
        _edge1_body,
        grid=(E // _TE,),
        in_specs=([pl.BlockSpec((_TE, 3 * HID), lambda i: (i, 0)),
                   pl.BlockSpec((_TE, 2 * HID), lambda i: (i, 0)),
                   pl.BlockSpec((_TE, EF), lambda i: (i, 0))]
                  + _mlp_specs(340) + _mlp_specs(340)
                  + [_full((1, RFEAT)), _full((1, 1)),
                     _full((EF, RFEAT)), _full((NRG, RFEAT)),
                     _full((HID, NH)), _full((NH, HID)),
                     _full((1, NRG))]),
        out_specs=[pl.BlockSpec((_TE, HID), lambda i: (i, 0)),
                   pl.BlockSpec((_TE, HID), lambda i: (i, 0))],
        out_shape=[jax.ShapeDtypeStruct((E, HID), jnp.float32),
                   jax.ShapeDtypeStruct((E, HID), jnp.float32)],
    )(g1d, g1s, ea, *_mlp_args_bf(pk), *_mlp_args_bf(pv),
      eww[None, :, 0], ewb[None, :], _A4, _A20, _S_HD, _S_WIDE, _OFFS)


def _node_mid(acc1, h, po, pq):
    return pl.pallas_call(
        _node_mid_body,
        grid=(N // _TN,),
        in_specs=([pl.BlockSpec((2, _TN, HID), lambda i: (0, i, 0)),
                   pl.BlockSpec((_TN, HID), lambda i: (i, 0))]
                  + _mlp_specs(2 * HID) + _mlp_specs(HID)
                  + [_full((NH, HID))]),
        out_specs=[pl.BlockSpec((_TN, HID), lambda i: (i, 0)),
                   pl.BlockSpec((_TN, 2 * HID), lambda i: (i, 0))],
        out_shape=[jax.ShapeDtypeStruct((N, HID), jnp.float32),
                   jax.ShapeDtypeStruct((N, 2 * HID), jnp.float32)],
    )(acc1, h, po["W1"], po["b1"][None, :], po["g"][None, :],
      po["be"][None, :], po["W2"], po["b2"][None, :], *_mlp_args(pq), _S_WIDE)


def _edge2(g2d, g2s, ea, rf, pk, pv, eww, ewb):
    vspecs = [_full((340, HID)), _full((1, HID)), _full((1, HID)),
              _full((1, HID)), _full((HID, NH)), _full((1, NH))]
    return pl.pallas_call(
        _edge2_body,
        grid=(E // _TE,),
        in_specs=([pl.BlockSpec((_TE, 2 * HID), lambda i: (i, 0)),
                   pl.BlockSpec((_TE, HID), lambda i: (i, 0)),
                   pl.BlockSpec((_TE, EF), lambda i: (i, 0)),
                   pl.BlockSpec((_TE, HID), lambda i: (i, 0))]
                  + _mlp_specs(340) + vspecs
                  + [_full((1, RFEAT)), _full((1, 1)),
                     _full((EF, RFEAT)), _full((NRG, RFEAT)),
                     _full((HID, NH)), _full((NH, 3 * NH)),
                     _full((3, 3 * NH))]),
        out_specs=[pl.BlockSpec((_TE, HID), lambda i: (i, 0))],
        out_shape=[jax.ShapeDtypeStruct((E, HID), jnp.float32)],
    )(g2d, g2s, ea, rf, *_mlp_args_bf(pk), *_mlp_args_bf(pv),
      eww[None, :, 0], ewb[None, :], _A4, _A20, _S_HD, _E1, _E2)[0]


def _node_fin(acc2):
    return pl.pallas_call(
        _node_fin_body,
        grid=(N // _TN,),
        in_specs=[pl.BlockSpec((2, _TN, HID), lambda i: (0, i, 0)),
                  _full((NH, 3 * NH)), _full((3 * NH, 3))],
        out_specs=[pl.BlockSpec((_TN, 3), lambda i: (i, 0))],
        out_shape=[jax.ShapeDtypeStruct((N, 3), jnp.float32)],
    )(acc2, _E1, _M48)[0]


_MESH = dict(core_axis_name="c", subcore_axis_name="s",
             num_cores=_NC, num_subcores=_NS)


@functools.lru_cache(maxsize=None)
def _make_gather(d1, d2, k):
    nchunk = E // k
    rounds = -(-nchunk // _NW)
    rounds2 = -(-rounds // 2)

    @functools.partial(
        pl.kernel,
        out_type=(jax.ShapeDtypeStruct((E, d1), jnp.float32),
                  jax.ShapeDtypeStruct((E, d2), jnp.float32)),
        mesh=plsc.VectorSubcoreMesh(**_MESH),
        scratch_types=[pltpu.VMEM((2, k), jnp.int32),
                       pltpu.VMEM((2, k), jnp.int32),
                       pltpu.VMEM((2, k, d1), jnp.float32),
                       pltpu.VMEM((2, k, d2), jnp.float32),
                       pltpu.SemaphoreType.DMA,
                       pltpu.SemaphoreType.DMA,
                       pltpu.SemaphoreType.DMA,
                       pltpu.SemaphoreType.DMA],
    )
    def g(t1, t2, dst, src, o1, o2, idv, isv, r1v, r2v, s1a, s1b, s2a, s2b):
        wid = lax.axis_index("s") * _NC + lax.axis_index("c")
        sem1 = (s1a, s1b)
        sem2 = (s2a, s2b)

        def cbase(r):
            cid = jnp.minimum(wid + r * _NW, nchunk - 1)
            return cid * k

        def issue(r, b):
            base = cbase(r)
            pltpu.sync_copy(dst.at[pl.ds(base, k)], idv.at[b])
            pltpu.sync_copy(src.at[pl.ds(base, k)], isv.at[b])
            pltpu.async_copy(t1.at[idv.at[b]], r1v.at[b], sem1[b])
            pltpu.async_copy(t2.at[isv.at[b]], r2v.at[b], sem2[b])

        def drain(r, b):
            pltpu.make_async_copy(t1.at[pl.ds(0, k)], r1v.at[b], sem1[b]).wait()
            pltpu.make_async_copy(t2.at[pl.ds(0, k)], r2v.at[b], sem2[b]).wait()
            base = cbase(r)
            pltpu.sync_copy(r1v.at[b], o1.at[pl.ds(base, k)])
            pltpu.sync_copy(r2v.at[b], o2.at[pl.ds(base, k)])

        issue(0, 0)

        def body(rr, carry):
            r = rr * 2
            issue(r + 1, 1)
            drain(r, 0)
            issue(r + 2, 0)
            drain(r + 1, 1)
            return carry

        lax.fori_loop(0, rounds2, body, 0)
        drain(2 * rounds2, 0)

    return g


@functools.lru_cache(maxsize=None)
def _make_scatter2arr():
    @functools.partial(
        pl.kernel,
        out_type=jax.ShapeDtypeStruct((_NC, N, HID), jnp.float32),
        mesh=plsc.VectorSubcoreMesh(**_MESH),
        scratch_types=[pltpu.VMEM((2, _K), jnp.int32),
                       pltpu.VMEM((2, _K, HID), jnp.float32),
                       pltpu.VMEM_SHARED((N, HID), jnp.float32),
                       pltpu.SemaphoreType.DMA,
                       pltpu.SemaphoreType.DMA],
    )
    def s(upda, updb, idx, zeros_hbm, out, idxv, updv, accum, sa, sb):
        cid = lax.axis_index("c")
        sid = lax.axis_index("s")
        sem = (sa, sb)

        @pl.when(sid == 0)
        def _():
            pltpu.sync_copy(zeros_hbm, accum)

        plsc.subcore_barrier()

        def run_for(upd):
            def load(r, b):
                ch = sid + r * _NS

                @pl.when(ch < _NCHUNK)
                def _():
                    base = ch * _K
                    pltpu.sync_copy(idx.at[pl.ds(base, _K)], idxv.at[b])
                    pltpu.async_copy(upd.at[pl.ds(base, _K)], updv.at[b],
                                     sem[b])

            def add(r, b):
                ch = sid + r * _NS

                @pl.when(ch < _NCHUNK)
                def _():
                    pltpu.make_async_copy(upd.at[pl.ds(0, _K)], updv.at[b],
                                          sem[b]).wait()
                    pltpu.sync_copy(updv.at[b], accum.at[idxv.at[b]],
                                    add=True)

            load(0, 0)

            def body(rr, carry):
                r = rr * 2
                load(r + 1, 1)
                add(r, 0)
                load(r + 2, 0)
                add(r + 1, 1)
                return carry

            r2 = -(-_ROUNDS16 // 2)
            lax.fori_loop(0, r2, body, 0)
            add(2 * r2, 0)

        @pl.when(cid == 0)
        def _():
            run_for(upda)

        @pl.when(cid == 1)
        def _():
            run_for(updb)

        plsc.subcore_barrier()

        @pl.when(sid == 0)
        def _():
            pltpu.sync_copy(accum, out.at[cid])

    def call(upda, updb, idx):
        return s(upda, updb, idx, jnp.zeros((N, HID), jnp.float32))

    return call


@functools.lru_cache(maxsize=None)
def _make_scatter1arr():
    @functools.partial(
        pl.kernel,
        out_type=jax.ShapeDtypeStruct((_NC, N, HID), jnp.float32),
        mesh=plsc.VectorSubcoreMesh(**_MESH),
        scratch_types=[pltpu.VMEM((2, _K), jnp.int32),
                       pltpu.VMEM((2, _K, HID), jnp.float32),
                       pltpu.VMEM_SHARED((N, HID), jnp.float32),
                       pltpu.SemaphoreType.DMA,
                       pltpu.SemaphoreType.DMA],
    )
    def s(upd, idx, zeros_hbm, out, idxv, updv, accum, sa, sb):
        cid = lax.axis_index("c")
        sid = lax.axis_index("s")
        wid = sid * _NC + cid
        sem = (sa, sb)

        @pl.when(sid == 0)
        def _():
            pltpu.sync_copy(zeros_hbm, accum)

        plsc.subcore_barrier()

        def load(r, b):
            ch = wid + r * _NW

            @pl.when(ch < _NCHUNK)
            def _():
                base = ch * _K
                pltpu.sync_copy(idx.at[pl.ds(base, _K)], idxv.at[b])
                pltpu.async_copy(upd.at[pl.ds(base, _K)], updv.at[b], sem[b])

        def add(r, b):
            ch = wid + r * _NW

            @pl.when(ch < _NCHUNK)
            def _():
                pltpu.make_async_copy(upd.at[pl.ds(0, _K)], updv.at[b],
                                      sem[b]).wait()
                pltpu.sync_copy(updv.at[b], accum.at[idxv.at[b]], add=True)

        load(0, 0)

        def body(rr, carry):
            r = rr * 2
            load(r + 1, 1)
            add(r, 0)
            load(r + 2, 0)
            add(r + 1, 1)
            return carry

        r2 = -(-_ROUNDS // 2)
        lax.fori_loop(0, r2, body, 0)
        add(2 * r2, 0)
        plsc.subcore_barrier()

        @pl.when(sid == 0)
        def _():
            pltpu.sync_copy(accum, out.at[cid])

    def call(upd, idx):
        return s(upd, idx, jnp.zeros((N, HID), jnp.float32))

    return call


def _gather1(t1d, t1s, dst, src):
    return _make_gather(3 * HID, 2 * HID, 64)(t1d, t1s, dst, src)


def _gather2(t2d, nh, dst, src):
    return _make_gather(2 * HID, HID, 128)(t2d, nh, dst, src)


def _scatter1(sca, scb, dst):
    return _make_scatter2arr()(sca, scb, dst)


def _scatter2(sc2, dst):
    return _make_scatter1arr()(sc2, dst)


def kernel(h, x, edge_attr, params, edge_index, mask_ligand):
    src = edge_index[0].astype(jnp.int32)
    dst = edge_index[1].astype(jnp.int32)
    p1 = params["x2h"]
    p2 = params["h2x"]
    xp = jnp.concatenate([x, jnp.zeros((N, HID - 3), jnp.float32)], axis=-1)

    t1d, t1s = _node_pre(h, xp, p1["hq"])
    g1d, g1s = _gather1(t1d, t1s, dst, src)
    sca, scb = _edge1(g1d, g1s, edge_attr, p1["hk"], p1["hv"],
                      p1["ew_W"], p1["ew_b"])
    acc1 = _scatter1(sca, scb, dst)
    x2h_out, t2d = _node_mid(acc1, h, p1["out"], p2["xq"])
    g2d, g2s = _gather2(t2d, x2h_out, dst, src)
    sc2 = _edge2(g2d, g2s, edge_attr, scb, p2["xk"], p2["xv"],
                 p2["ew_W"], p2["ew_b"])
    acc2 = _scatter2(sc2, dst)
    delta_x = _node_fin(acc2)
    return (x2h_out, delta_x)


# speedup vs baseline: 1.0933x; 1.0933x over previous
import functools

import numpy as np
import jax
import jax.numpy as jnp
from jax import lax
from jax.experimental import pallas as pl
from jax.experimental.pallas import tpu as pltpu
from jax.experimental.pallas import tpu_sc as plsc

N = 10000
E = 320000
HID = 128
NH = 16
HD = HID // NH
NRG = 20
EF = 4
RFEAT = NRG * EF
R_MIN, R_MAX = 0.0, 10.0
_STEP = (R_MAX - R_MIN) / (NRG - 1)
_COEFF = -0.5 / _STEP**2
_OFFS = np.linspace(R_MIN, R_MAX, NRG).astype(np.float32)[None, :]
_ISQ = np.float32(1.0 / np.sqrt(HD))

_NC, _NS = 2, 16
_NW = _NC * _NS
_K = 128
_NCHUNK = E // _K
_ROUNDS = -(-_NCHUNK // _NW)
_ROUNDS16 = -(-_NCHUNK // _NS)

_TE = 4000
_TN = 2000


def _sel(n_in, n_out, fn):
    m = np.zeros((n_in, n_out), np.float32)
    for i in range(n_in):
        for j in range(n_out):
            if fn(i, j):
                m[i, j] = 1.0
    return m


_S_HD = _sel(HID, NH, lambda j, h: j // HD == h)
_S_WIDE = _sel(NH, HID, lambda h, j: j // HD == h)
_A4 = _sel(EF, RFEAT, lambda f, j: j // NRG == f)
_A20 = _sel(NRG, RFEAT, lambda g, j: j % NRG == g)
_E1 = _sel(NH, 3 * NH, lambda h, j: j // 3 == h)
_E2 = _sel(3, 3 * NH, lambda c, j: j % 3 == c)
_M48 = _sel(3 * NH, 3, lambda j, c: j % 3 == c) / NH


def _ln_relu(h1, g, be):
    mu = jnp.mean(h1, axis=-1, keepdims=True)
    var = jnp.mean((h1 - mu) ** 2, axis=-1, keepdims=True)
    hn = (h1 - mu) * lax.rsqrt(var + 1e-5) * g + be
    return jnp.maximum(hn, 0.0)


def _dot(a, b):
    return jnp.dot(a, b, preferred_element_type=jnp.float32)


def _node_pre_body(h_ref, xp_ref, w1, b1, g, be, w2, b2, t1d, t1s):
    h = h_ref[:]
    xp = xp_ref[:]
    r = _ln_relu(_dot(h, w1[:]) + b1[:], g[:], be[:])
    q = _dot(r, w2[:]) + b2[:]
    t1d[:] = jnp.concatenate([h, q, xp], axis=-1)
    t1s[:] = jnp.concatenate([h, xp], axis=-1)


def _edge1_body(g1d, g1s, ea_ref,
                kw1, kb1, kg, kbe, kw2, kb2,
                vw1, vb1, vg, vbe, vw2, vb2,
                eww, ewb, a4, a20, shd, swide, offs,
                sca, scb):
    d = g1d[:]
    hi = d[:, 0:HID]
    qd = d[:, HID:2 * HID]
    s = g1s[:]
    hj = s[:, 0:HID]
    relp = d[:, 2 * HID:2 * HID + 16] - s[:, HID:HID + 16]
    rel = relp[:, 0:3]
    ea = ea_ref[:]
    dist = jnp.sqrt(jnp.sum(relp * relp, axis=-1, keepdims=True))
    df = jnp.exp(_COEFF * (dist - offs[:]) ** 2)
    rf = _dot(ea, a4[:]) * _dot(df, a20[:])
    kv84 = jnp.concatenate([ea, rf], axis=-1)
    h1k = (_dot(kv84, kw1[0:84, :]) + _dot(hi, kw1[84:212, :])
           + _dot(hj, kw1[212:340, :]) + kb1[:])
    k = _dot(_ln_relu(h1k, kg[:], kbe[:]), kw2[:]) + kb2[:]
    h1v = (_dot(kv84, vw1[0:84, :]) + _dot(hi, vw1[84:212, :])
           + _dot(hj, vw1[212:340, :]) + vb1[:])
    v = _dot(_ln_relu(h1v, vg[:], vbe[:]), vw2[:]) + vb2[:]
    ew = jax.nn.sigmoid(jnp.sum(rf * eww[:], axis=-1, keepdims=True) + ewb[:])
    v = v * ew
    w = jnp.exp(_dot(qd * k, shd[:]) * _ISQ)
    wv = _dot(w, swide[:]) * v
    sca[:] = wv
    zpad = jnp.zeros((w.shape[0], HID - NH - 4 - NRG), jnp.float32)
    scb[:] = jnp.concatenate([w, rel, dist, df, zpad], axis=-1)


def _node_mid_body(acc_ref, h_ref,
                   ow1, ob1, og, obe, ow2, ob2,
                   qw1, qb1, qg, qbe, qw2, qb2,
                   swide, nh_out, t2d):
    num = acc_ref[0]
    den = acc_ref[1][:, 0:NH]
    attn = num / (_dot(den, swide[:]) + 1e-16)
    h = h_ref[:]
    h1 = _dot(attn, ow1[0:HID, :]) + _dot(h, ow1[HID:2 * HID, :]) + ob1[:]
    o = _dot(_ln_relu(h1, og[:], obe[:]), ow2[:]) + ob2[:]
    nh = o + h
    r = _ln_relu(_dot(nh, qw1[:]) + qb1[:], qg[:], qbe[:])
    q2 = _dot(r, qw2[:]) + qb2[:]
    nh_out[:] = nh
    t2d[:] = jnp.concatenate([nh, q2], axis=-1)


def _edge2_body(g2d, g2s, ea_ref, rf_ref,
                kw1, kb1, kg, kbe, kw2, kb2,
                vw1, vb1, vg, vbe, vw2, vb2,
                eww, ewb, a4, a20, shd, e1, e2,
                sc2):
    d = g2d[:]
    hi = d[:, 0:HID]
    qd = d[:, HID:2 * HID]
    hj = g2s[:]
    ea = ea_ref[:]
    rfin = rf_ref[:]
    rel = rfin[:, NH:NH + 3]
    df = rfin[:, NH + 4:NH + 4 + NRG]
    rf = _dot(ea, a4[:]) * _dot(df, a20[:])
    kv84 = jnp.concatenate([ea, rf], axis=-1)
    h1k = (_dot(kv84, kw1[0:84, :]) + _dot(hi, kw1[84:212, :])
           + _dot(hj, kw1[212:340, :]) + kb1[:])
    k = _dot(_ln_relu(h1k, kg[:], kbe[:]), kw2[:]) + kb2[:]
    h1v = (_dot(kv84, vw1[0:84, :]) + _dot(hi, vw1[84:212, :])
           + _dot(hj, vw1[212:340, :]) + vb1[:])
    v2 = _dot(_ln_relu(h1v, vg[:], vbe[:]), vw2[:]) + vb2[:]
    ew = jax.nn.sigmoid(jnp.sum(rf * eww[:], axis=-1, keepdims=True) + ewb[:])
    v2 = v2 * ew
    w = jnp.exp(_dot(qd * k, shd[:]) * _ISQ)
    mv = w * v2
    sv = _dot(mv, e1[:]) * _dot(rel, e2[:])
    zpad = jnp.zeros((w.shape[0], HID - 4 * NH), jnp.float32)
    sc2[:] = jnp.concatenate([sv, w, zpad], axis=-1)


def _node_fin_body(acc_ref, e1, m48, delta):
    a = acc_ref[0] + acc_ref[1]
    num = a[:, 0:3 * NH]
    den = a[:, 3 * NH:4 * NH]
    ratio = num / (_dot(den, e1[:]) + 1e-16)
    delta[:] = _dot(ratio, m48[:])


def _full(shape):
    return pl.BlockSpec(shape, lambda i: (0,) * len(shape))


def _mlp_specs(din):
    return [_full((din, HID)), _full((1, HID)), _full((1, HID)),
            _full((1, HID)), _full((HID, HID)), _full((1, HID))]


def _mlp_args(p):
    return (p["W1"], p["b1"][None, :], p["g"][None, :], p["be"][None, :],
            p["W2"], p["b2"][None, :])


def _node_pre(h, xp, p):
    return pl.pallas_call(
        _node_pre_body,
        grid=(N // _TN,),
        in_specs=([pl.BlockSpec((_TN, HID), lambda i: (i, 0)),
                   pl.BlockSpec((_TN, HID), lambda i: (i, 0))]
                  + _mlp_specs(HID)),
        out_specs=[pl.BlockSpec((_TN, 3 * HID), lambda i: (i, 0)),
                   pl.BlockSpec((_TN, 2 * HID), lambda i: (i, 0))],
        out_shape=[jax.ShapeDtypeStruct((N, 3 * HID), jnp.float32),
                   jax.ShapeDtypeStruct((N, 2 * HID), jnp.float32)],
    )(h, xp, *_mlp_args(p))


def _edge1(g1d, g1s, ea, pk, pv, eww, ewb):
    return pl.pallas_call(
        _edge1_body,
        grid=(E // _TE,),
        in_specs=([pl.BlockSpec((_TE, 3 * HID), lambda i: (i, 0)),
                   pl.BlockSpec((_TE, 2 * HID), lambda i: (i, 0)),
                   pl.BlockSpec((_TE, EF), lambda i: (i, 0))]
                  + _mlp_specs(340) + _mlp_specs(340)
                  + [_full((1, RFEAT)), _full((1, 1)),
                     _full((EF, RFEAT)), _full((NRG, RFEAT)),
                     _full((HID, NH)), _full((NH, HID)),
                     _full((1, NRG))]),
        out_specs=[pl.BlockSpec((_TE, HID), lambda i: (i, 0)),
                   pl.BlockSpec((_TE, HID), lambda i: (i, 0))],
        out_shape=[jax.ShapeDtypeStruct((E, HID), jnp.float32),
                   jax.ShapeDtypeStruct((E, HID), jnp.float32)],
    )(g1d, g1s, ea, *_mlp_args(pk), *_mlp_args(pv),
      eww[None, :, 0], ewb[None, :], _A4, _A20, _S_HD, _S_WIDE, _OFFS)


def _node_mid(acc1, h, po, pq):
    return pl.pallas_call(
        _node_mid_body,
        grid=(N // _TN,),
        in_specs=([pl.BlockSpec((2, _TN, HID), lambda i: (0, i, 0)),
                   pl.BlockSpec((_TN, HID), lambda i: (i, 0))]
                  + _mlp_specs(2 * HID) + _mlp_specs(HID)
                  + [_full((NH, HID))]),
        out_specs=[pl.BlockSpec((_TN, HID), lambda i: (i, 0)),
                   pl.BlockSpec((_TN, 2 * HID), lambda i: (i, 0))],
        out_shape=[jax.ShapeDtypeStruct((N, HID), jnp.float32),
                   jax.ShapeDtypeStruct((N, 2 * HID), jnp.float32)],
    )(acc1, h, po["W1"], po["b1"][None, :], po["g"][None, :],
      po["be"][None, :], po["W2"], po["b2"][None, :], *_mlp_args(pq), _S_WIDE)


def _edge2(g2d, g2s, ea, rf, pk, pv, eww, ewb):
    vspecs = [_full((340, HID)), _full((1, HID)), _full((1, HID)),
              _full((1, HID)), _full((HID, NH)), _full((1, NH))]
    return pl.pallas_call(
        _edge2_body,
        grid=(E // _TE,),
        in_specs=([pl.BlockSpec((_TE, 2 * HID), lambda i: (i, 0)),
                   pl.BlockSpec((_TE, HID), lambda i: (i, 0)),
                   pl.BlockSpec((_TE, EF), lambda i: (i, 0)),
                   pl.BlockSpec((_TE, HID), lambda i: (i, 0))]
                  + _mlp_specs(340) + vspecs
                  + [_full((1, RFEAT)), _full((1, 1)),
                     _full((EF, RFEAT)), _full((NRG, RFEAT)),
                     _full((HID, NH)), _full((NH, 3 * NH)),
                     _full((3, 3 * NH))]),
        out_specs=[pl.BlockSpec((_TE, HID), lambda i: (i, 0))],
        out_shape=[jax.ShapeDtypeStruct((E, HID), jnp.float32)],
    )(g2d, g2s, ea, rf, *_mlp_args(pk), *_mlp_args(pv),
      eww[None, :, 0], ewb[None, :], _A4, _A20, _S_HD, _E1, _E2)[0]


def _node_fin(acc2):
    return pl.pallas_call(
        _node_fin_body,
        grid=(N // _TN,),
        in_specs=[pl.BlockSpec((2, _TN, HID), lambda i: (0, i, 0)),
                  _full((NH, 3 * NH)), _full((3 * NH, 3))],
        out_specs=[pl.BlockSpec((_TN, 3), lambda i: (i, 0))],
        out_shape=[jax.ShapeDtypeStruct((N, 3), jnp.float32)],
    )(acc2, _E1, _M48)[0]


_MESH = dict(core_axis_name="c", subcore_axis_name="s",
             num_cores=_NC, num_subcores=_NS)


@functools.lru_cache(maxsize=None)
def _make_gather(d1, d2, k):
    nchunk = E // k
    rounds = -(-nchunk // _NW)
    rounds2 = -(-rounds // 2)

    @functools.partial(
        pl.kernel,
        out_type=(jax.ShapeDtypeStruct((E, d1), jnp.float32),
                  jax.ShapeDtypeStruct((E, d2), jnp.float32)),
        mesh=plsc.VectorSubcoreMesh(**_MESH),
        scratch_types=[pltpu.VMEM((2, k), jnp.int32),
                       pltpu.VMEM((2, k), jnp.int32),
                       pltpu.VMEM((2, k, d1), jnp.float32),
                       pltpu.VMEM((2, k, d2), jnp.float32),
                       pltpu.SemaphoreType.DMA,
                       pltpu.SemaphoreType.DMA,
                       pltpu.SemaphoreType.DMA,
                       pltpu.SemaphoreType.DMA],
    )
    def g(t1, t2, dst, src, o1, o2, idv, isv, r1v, r2v, s1a, s1b, s2a, s2b):
        wid = lax.axis_index("s") * _NC + lax.axis_index("c")
        sem1 = (s1a, s1b)
        sem2 = (s2a, s2b)

        def cbase(r):
            cid = jnp.minimum(wid + r * _NW, nchunk - 1)
            return cid * k

        def issue(r, b):
            base = cbase(r)
            pltpu.sync_copy(dst.at[pl.ds(base, k)], idv.at[b])
            pltpu.sync_copy(src.at[pl.ds(base, k)], isv.at[b])
            pltpu.async_copy(t1.at[idv.at[b]], r1v.at[b], sem1[b])
            pltpu.async_copy(t2.at[isv.at[b]], r2v.at[b], sem2[b])

        def drain(r, b):
            pltpu.make_async_copy(t1.at[pl.ds(0, k)], r1v.at[b], sem1[b]).wait()
            pltpu.make_async_copy(t2.at[pl.ds(0, k)], r2v.at[b], sem2[b]).wait()
            base = cbase(r)
            pltpu.sync_copy(r1v.at[b], o1.at[pl.ds(base, k)])
            pltpu.sync_copy(r2v.at[b], o2.at[pl.ds(base, k)])

        issue(0, 0)

        def body(rr, carry):
            r = rr * 2
            issue(r + 1, 1)
            drain(r, 0)
            issue(r + 2, 0)
            drain(r + 1, 1)
            return carry

        lax.fori_loop(0, rounds2, body, 0)
        drain(2 * rounds2, 0)

    return g


@functools.lru_cache(maxsize=None)
def _make_scatter2arr():
    @functools.partial(
        pl.kernel,
        out_type=jax.ShapeDtypeStruct((_NC, N, HID), jnp.float32),
        mesh=plsc.VectorSubcoreMesh(**_MESH),
        scratch_types=[pltpu.VMEM((2, _K), jnp.int32),
                       pltpu.VMEM((2, _K, HID), jnp.float32),
                       pltpu.VMEM_SHARED((N, HID), jnp.float32),
                       pltpu.SemaphoreType.DMA,
                       pltpu.SemaphoreType.DMA],
    )
    def s(upda, updb, idx, zeros_hbm, out, idxv, updv, accum, sa, sb):
        cid = lax.axis_index("c")
        sid = lax.axis_index("s")
        sem = (sa, sb)

        @pl.when(sid == 0)
        def _():
            pltpu.sync_copy(zeros_hbm, accum)

        plsc.subcore_barrier()

        def run_for(upd):
            def load(r, b):
                ch = sid + r * _NS

                @pl.when(ch < _NCHUNK)
                def _():
                    base = ch * _K
                    pltpu.sync_copy(idx.at[pl.ds(base, _K)], idxv.at[b])
                    pltpu.async_copy(upd.at[pl.ds(base, _K)], updv.at[b],
                                     sem[b])

            def add(r, b):
                ch = sid + r * _NS

                @pl.when(ch < _NCHUNK)
                def _():
                    pltpu.make_async_copy(upd.at[pl.ds(0, _K)], updv.at[b],
                                          sem[b]).wait()
                    pltpu.sync_copy(updv.at[b], accum.at[idxv.at[b]],
                                    add=True)

            load(0, 0)

            def body(rr, carry):
                r = rr * 2
                load(r + 1, 1)
                add(r, 0)
                load(r + 2, 0)
                add(r + 1, 1)
                return carry

            r2 = -(-_ROUNDS16 // 2)
            lax.fori_loop(0, r2, body, 0)
            add(2 * r2, 0)

        @pl.when(cid == 0)
        def _():
            run_for(upda)

        @pl.when(cid == 1)
        def _():
            run_for(updb)

        plsc.subcore_barrier()

        @pl.when(sid == 0)
        def _():
            pltpu.sync_copy(accum, out.at[cid])

    def call(upda, updb, idx):
        return s(upda, updb, idx, jnp.zeros((N, HID), jnp.float32))

    return call


@functools.lru_cache(maxsize=None)
def _make_scatter1arr():
    @functools.partial(
        pl.kernel,
        out_type=jax.ShapeDtypeStruct((_NC, N, HID), jnp.float32),
        mesh=plsc.VectorSubcoreMesh(**_MESH),
        scratch_types=[pltpu.VMEM((2, _K), jnp.int32),
                       pltpu.VMEM((2, _K, HID), jnp.float32),
                       pltpu.VMEM_SHARED((N, HID), jnp.float32),
                       pltpu.SemaphoreType.DMA,
                       pltpu.SemaphoreType.DMA],
    )
    def s(upd, idx, zeros_hbm, out, idxv, updv, accum, sa, sb):
        cid = lax.axis_index("c")
        sid = lax.axis_index("s")
        wid = sid * _NC + cid
        sem = (sa, sb)

        @pl.when(sid == 0)
        def _():
            pltpu.sync_copy(zeros_hbm, accum)

        plsc.subcore_barrier()

        def load(r, b):
            ch = wid + r * _NW

            @pl.when(ch < _NCHUNK)
            def _():
                base = ch * _K
                pltpu.sync_copy(idx.at[pl.ds(base, _K)], idxv.at[b])
                pltpu.async_copy(upd.at[pl.ds(base, _K)], updv.at[b], sem[b])

        def add(r, b):
            ch = wid + r * _NW

            @pl.when(ch < _NCHUNK)
            def _():
                pltpu.make_async_copy(upd.at[pl.ds(0, _K)], updv.at[b],
                                      sem[b]).wait()
                pltpu.sync_copy(updv.at[b], accum.at[idxv.at[b]], add=True)

        load(0, 0)

        def body(rr, carry):
            r = rr * 2
            load(r + 1, 1)
            add(r, 0)
            load(r + 2, 0)
            add(r + 1, 1)
            return carry

        r2 = -(-_ROUNDS // 2)
        lax.fori_loop(0, r2, body, 0)
        add(2 * r2, 0)
        plsc.subcore_barrier()

        @pl.when(sid == 0)
        def _():
            pltpu.sync_copy(accum, out.at[cid])

    def call(upd, idx):
        return s(upd, idx, jnp.zeros((N, HID), jnp.float32))

    return call


def _gather1(t1d, t1s, dst, src):
    return _make_gather(3 * HID, 2 * HID, 64)(t1d, t1s, dst, src)


def _gather2(t2d, nh, dst, src):
    return _make_gather(2 * HID, HID, 128)(t2d, nh, dst, src)


def _scatter1(sca, scb, dst):
    return _make_scatter2arr()(sca, scb, dst)


def _scatter2(sc2, dst):
    return _make_scatter1arr()(sc2, dst)


def kernel(h, x, edge_attr, params, edge_index, mask_ligand):
    src = edge_index[0].astype(jnp.int32)
    dst = edge_index[1].astype(jnp.int32)
    p1 = params["x2h"]
    p2 = params["h2x"]
    xp = jnp.concatenate([x, jnp.zeros((N, HID - 3), jnp.float32)], axis=-1)

    t1d, t1s = _node_pre(h, xp, p1["hq"])
    g1d, g1s = _gather1(t1d, t1s, dst, src)
    sca, scb = _edge1(g1d, g1s, edge_attr, p1["hk"], p1["hv"],
                      p1["ew_W"], p1["ew_b"])
    acc1 = _scatter1(sca, scb, dst)
    x2h_out, t2d = _node_mid(acc1, h, p1["out"], p2["xq"])
    g2d, g2s = _gather2(t2d, x2h_out, dst, src)
    sc2 = _edge2(g2d, g2s, edge_attr, scb, p2["xk"], p2["xv"],
                 p2["ew_W"], p2["ew_b"])
    acc2 = _scatter2(sc2, dst)
    delta_x = _node_fin(acc2)
    return (x2h_out, delta_x)
